# baseline (speedup 1.0000x reference)
import functools

import jax
import jax.numpy as jnp
from jax import lax
from jax.experimental import pallas as pl
from jax.experimental.pallas import tpu as pltpu
from jax.experimental.pallas import tpu_sc as plsc

H = 768
S = 512
EPS = 1e-12

_NC = 2
_NS = 16
_NW = _NC * _NS


def _sc_gather(word_emb, ids_flat):
    n = ids_flat.shape[0]
    rpw = n // _NW
    g = 64
    mesh = plsc.VectorSubcoreMesh(core_axis_name="c", subcore_axis_name="s",
                                  num_cores=_NC, num_subcores=_NS)

    @functools.partial(
        pl.kernel,
        out_type=jax.ShapeDtypeStruct((n, H), jnp.float32),
        mesh=mesh,
        scratch_types=[
            pltpu.VMEM((g,), jnp.int32),
            pltpu.VMEM((g, H), jnp.float32),
            pltpu.SemaphoreType.DMA,
        ],
    )
    def gather_kernel(table_hbm, idx_hbm, out_hbm, idx_v, rows_v, sem):
        wid = lax.axis_index("s") * _NC + lax.axis_index("c")
        base = wid * rpw

        def body(i, carry):
            off = pl.multiple_of(base + i * g, g)
            pltpu.sync_copy(idx_hbm.at[pl.ds(off, g)], idx_v)
            pltpu.async_copy(table_hbm.at[idx_v], rows_v, sem).wait()
            pltpu.sync_copy(rows_v, out_hbm.at[pl.ds(off, g)])
            return carry

        lax.fori_loop(0, rpw // g, body, 0)

    return gather_kernel(word_emb, ids_flat)


def _tc_combine(we_flat, text_len, pe_plus, consts):
    n = we_flat.shape[0]
    blk = 64
    grid = (n // blk,)

    def body(tl_ref, we_ref, weany_ref, pe_ref, c_ref, out_ref, win_ref, sem):
        k = pl.program_id(0)
        b = k // (S // blk)
        s0 = (k % (S // blk)) * blk
        ln = tl_ref[b]
        c = jnp.maximum(ln - 6, 0)
        c8 = jnp.minimum((c // 8) * 8, S - 16)
        cp = pltpu.make_async_copy(
            weany_ref.at[pl.ds(b * S + c8, 16)], win_ref, sem)
        cp.start()
        cp.wait()
        p = c8 + lax.broadcasted_iota(jnp.int32, (16, 1), 0)
        valid = ((p >= c) & (p <= ln - 2)).astype(jnp.float32)
        avg = jnp.sum(win_ref[...] * valid, axis=0, keepdims=True) * 0.2
        svec = s0 + lax.broadcasted_iota(jnp.int32, (blk, 1), 0)
        sep = c_ref[0:1, :]
        pad = c_ref[1:2, :]
        gamma = c_ref[2:3, :]
        beta = c_ref[3:4, :]
        sel = jnp.where(svec < ln - 6, we_ref[...],
                        jnp.where(svec == ln - 6, avg,
                                  jnp.where(svec == ln - 5, sep, pad)))
        x = sel + pe_ref[...]
        mu = jnp.mean(x, axis=1, keepdims=True)
        xc = x - mu
        var = jnp.mean(xc * xc, axis=1, keepdims=True)
        y = xc * lax.rsqrt(var + EPS)
        out_ref[...] = y * gamma + beta

    return pl.pallas_call(
        body,
        grid=grid,
        in_specs=[
            pl.BlockSpec(memory_space=pltpu.SMEM),
            pl.BlockSpec((blk, H), lambda k: (k, 0)),
            pl.BlockSpec(memory_space=pl.ANY),
            pl.BlockSpec((blk, H), lambda k: (k % (S // blk), 0)),
            pl.BlockSpec((8, H), lambda k: (0, 0)),
        ],
        out_specs=pl.BlockSpec((blk, H), lambda k: (k, 0)),
        out_shape=jax.ShapeDtypeStruct((n, H), jnp.float32),
        scratch_shapes=[
            pltpu.VMEM((16, H), jnp.float32),
            pltpu.SemaphoreType.DMA,
        ],
    )(text_len, we_flat, we_flat, pe_plus, consts)


def kernel(input_ids, text_len, word_emb, pos_emb, type_emb, ln_gamma, ln_beta):
    b, s = input_ids.shape
    ids_flat = input_ids.reshape(-1).astype(jnp.int32)
    tl = text_len.astype(jnp.int32)
    pe_plus = pos_emb + type_emb[0][None, :]
    consts = jnp.concatenate(
        [word_emb[102:103], word_emb[0:1], ln_gamma[None, :], ln_beta[None, :],
         jnp.zeros((4, H), jnp.float32)], axis=0)
    we_flat = _sc_gather(word_emb, ids_flat)
    out = _tc_combine(we_flat, tl, pe_plus, consts)
    return out.reshape(b, s, H)


# speedup vs baseline: 1.7246x; 1.7246x over previous
import functools

import jax
import jax.numpy as jnp
from jax import lax
from jax.experimental import pallas as pl
from jax.experimental.pallas import tpu as pltpu
from jax.experimental.pallas import tpu_sc as plsc

H = 768
S = 512
EPS = 1e-12

_NC = 2
_NS = 16
_NW = _NC * _NS


def _sc_gather(word_emb, ids_flat, text_len):
    n = ids_flat.shape[0]
    nb = text_len.shape[0]
    rpw = n // _NW
    bpw = nb // _NW
    g = 64
    mesh = plsc.VectorSubcoreMesh(core_axis_name="c", subcore_axis_name="s",
                                  num_cores=_NC, num_subcores=_NS)

    @functools.partial(
        pl.kernel,
        out_type=(jax.ShapeDtypeStruct((n, H), jnp.float32),
                  jax.ShapeDtypeStruct((nb, H), jnp.float32)),
        mesh=mesh,
        scratch_types=[
            pltpu.VMEM((g,), jnp.int32),
            pltpu.VMEM((g, H), jnp.float32),
            pltpu.VMEM((16,), jnp.int32),
            pltpu.VMEM((16,), jnp.int32),
            pltpu.VMEM((16, H), jnp.float32),
            pltpu.VMEM((16,), jnp.int32),
            pltpu.VMEM((H,), jnp.float32),
            pltpu.SemaphoreType.DMA,
        ],
    )
    def gather_kernel(table_hbm, idx_hbm, tl_hbm, out_hbm, avg_hbm,
                      idx_v, rows_v, wids_v, widx_v, wrows_v, tl_v, avg_v,
                      sem):
        wid = lax.axis_index("s") * _NC + lax.axis_index("c")
        base = wid * rpw

        def body(i, carry):
            off = pl.multiple_of(base + i * g, g)
            pltpu.sync_copy(idx_hbm.at[pl.ds(off, g)], idx_v)
            pltpu.async_copy(table_hbm.at[idx_v], rows_v, sem).wait()
            pltpu.sync_copy(rows_v, out_hbm.at[pl.ds(off, g)])
            return carry

        lax.fori_loop(0, rpw // g, body, 0)

        iota = lax.broadcasted_iota(jnp.int32, (16,), 0)
        pltpu.sync_copy(tl_hbm.at[pl.ds(wid * bpw, bpw)], tl_v.at[pl.ds(0, bpw)])
        tl_vec = tl_v[...]
        for j in range(bpw):
            b = wid * bpw + j
            ln = tl_vec[j]
            c = jnp.maximum(ln - 6, 0)
            c8 = jnp.minimum((c // 8) * 8, S - 16)
            pltpu.sync_copy(idx_hbm.at[pl.ds(b * S + c8, 16)], wids_v)
            gidx = jnp.minimum((c - c8) + iota, 15)
            widx_v[...] = wids_v[...].at[gidx].get(mode="promise_in_bounds")
            pltpu.async_copy(table_hbm.at[widx_v], wrows_v, sem).wait()
            nlast = ln - 2 - c

            def kbody(k, carry):
                koff = pl.multiple_of(k * 16, 16)
                acc = jnp.zeros((16,), jnp.float32)
                for j2 in range(5):
                    w = jnp.where(nlast >= j2, 0.2, 0.0)
                    acc = acc + wrows_v[j2, pl.ds(koff, 16)] * w
                avg_v[pl.ds(koff, 16)] = acc
                return carry

            lax.fori_loop(0, H // 16, kbody, 0)
            pltpu.sync_copy(avg_v, avg_hbm.at[b])

    return gather_kernel(word_emb, ids_flat, text_len)


def _tc_combine(we_flat, avg, text_len, pe_plus, consts):
    n = we_flat.shape[0]
    blk = 64
    grid = (n // blk,)
    sb = S // blk

    def body(tl_ref, we_ref, avg_ref, pe_ref, c_ref, out_ref):
        k = pl.program_id(0)
        b = k // sb
        s0 = (k % sb) * blk
        ln = tl_ref[b]
        svec = s0 + lax.broadcasted_iota(jnp.int32, (blk, 1), 0)
        sep = c_ref[0:1, :]
        pad = c_ref[1:2, :]
        gamma = c_ref[2:3, :]
        beta = c_ref[3:4, :]
        sel = jnp.where(svec < ln - 6, we_ref[...],
                        jnp.where(svec == ln - 6, avg_ref[0],
                                  jnp.where(svec == ln - 5, sep, pad)))
        x = sel + pe_ref[...]
        mu = jnp.mean(x, axis=1, keepdims=True)
        xc = x - mu
        var = jnp.mean(xc * xc, axis=1, keepdims=True)
        y = xc * lax.rsqrt(var + EPS)
        out_ref[...] = y * gamma + beta

    return pl.pallas_call(
        body,
        grid=grid,
        in_specs=[
            pl.BlockSpec(memory_space=pltpu.SMEM),
            pl.BlockSpec((blk, H), lambda k: (k, 0)),
            pl.BlockSpec((1, 1, H), lambda k: (k // sb, 0, 0)),
            pl.BlockSpec((blk, H), lambda k: (k % sb, 0)),
            pl.BlockSpec((8, H), lambda k: (0, 0)),
        ],
        out_specs=pl.BlockSpec((blk, H), lambda k: (k, 0)),
        out_shape=jax.ShapeDtypeStruct((n, H), jnp.float32),
    )(text_len, we_flat, avg.reshape(-1, 1, H), pe_plus, consts)


def kernel(input_ids, text_len, word_emb, pos_emb, type_emb, ln_gamma, ln_beta):
    b, s = input_ids.shape
    ids_flat = input_ids.reshape(-1).astype(jnp.int32)
    tl = text_len.astype(jnp.int32)
    pe_plus = pos_emb + type_emb[0][None, :]
    consts = jnp.concatenate(
        [word_emb[102:103], word_emb[0:1], ln_gamma[None, :], ln_beta[None, :],
         jnp.zeros((4, H), jnp.float32)], axis=0)
    we_flat, avg = _sc_gather(word_emb, ids_flat, tl)
    out = _tc_combine(we_flat, avg, tl, pe_plus, consts)
    return out.reshape(b, s, H)


# speedup vs baseline: 1.8090x; 1.0490x over previous
import functools

import jax
import jax.numpy as jnp
from jax import lax
from jax.experimental import pallas as pl
from jax.experimental.pallas import tpu as pltpu
from jax.experimental.pallas import tpu_sc as plsc

H = 768
S = 512
EPS = 1e-12

_NC = 2
_NS = 16
_NW = _NC * _NS


def _sc_gather(word_emb, ids_flat, text_len):
    n = ids_flat.shape[0]
    nb = text_len.shape[0]
    rpw = n // _NW
    bpw = nb // _NW
    g = 64
    mesh = plsc.VectorSubcoreMesh(core_axis_name="c", subcore_axis_name="s",
                                  num_cores=_NC, num_subcores=_NS)

    @functools.partial(
        pl.kernel,
        out_type=(jax.ShapeDtypeStruct((n, H), jnp.float32),
                  jax.ShapeDtypeStruct((nb, H), jnp.float32)),
        mesh=mesh,
        scratch_types=[
            pltpu.VMEM((g,), jnp.int32),
            pltpu.VMEM((g, H), jnp.float32),
            pltpu.VMEM((16,), jnp.int32),
            pltpu.VMEM((16,), jnp.int32),
            pltpu.VMEM((16, H), jnp.float32),
            pltpu.VMEM((16,), jnp.int32),
            pltpu.VMEM((H,), jnp.float32),
            pltpu.SemaphoreType.DMA,
        ],
    )
    def gather_kernel(table_hbm, idx_hbm, tl_hbm, out_hbm, avg_hbm,
                      idx_v, rows_v, wids_v, widx_v, wrows_v, tl_v, avg_v,
                      sem):
        wid = lax.axis_index("s") * _NC + lax.axis_index("c")
        base = wid * rpw

        def body(i, carry):
            off = pl.multiple_of(base + i * g, g)
            pltpu.sync_copy(idx_hbm.at[pl.ds(off, g)], idx_v)
            pltpu.async_copy(table_hbm.at[idx_v], rows_v, sem).wait()
            pltpu.sync_copy(rows_v, out_hbm.at[pl.ds(off, g)])
            return carry

        lax.fori_loop(0, rpw // g, body, 0)

        iota = lax.broadcasted_iota(jnp.int32, (16,), 0)
        pltpu.sync_copy(tl_hbm.at[pl.ds(wid * bpw, bpw)], tl_v.at[pl.ds(0, bpw)])
        tl_vec = tl_v[...]
        for j in range(bpw):
            b = wid * bpw + j
            ln = tl_vec[j]
            c = jnp.maximum(ln - 6, 0)
            c8 = jnp.minimum((c // 8) * 8, S - 16)
            pltpu.sync_copy(idx_hbm.at[pl.ds(b * S + c8, 16)], wids_v)
            gidx = jnp.minimum((c - c8) + iota, 15)
            widx_v[...] = wids_v[...].at[gidx].get(mode="promise_in_bounds")
            pltpu.async_copy(table_hbm.at[widx_v], wrows_v, sem).wait()
            nlast = ln - 2 - c

            def kbody(k, carry):
                koff = pl.multiple_of(k * 16, 16)
                acc = jnp.zeros((16,), jnp.float32)
                for j2 in range(5):
                    w = jnp.where(nlast >= j2, 0.2, 0.0)
                    acc = acc + wrows_v[j2, pl.ds(koff, 16)] * w
                avg_v[pl.ds(koff, 16)] = acc
                return carry

            lax.fori_loop(0, H // 16, kbody, 0)
            pltpu.sync_copy(avg_v, avg_hbm.at[b])

    return gather_kernel(word_emb, ids_flat, text_len)


def _tc_combine(we_flat, avg, text_len, pe_plus, consts):
    n = we_flat.shape[0]
    blk = 64
    sb = S // blk
    nb = n // S
    grid = (sb, nb)

    def body(tl_ref, we_ref, avg_ref, pe_ref, c_ref, out_ref):
        j = pl.program_id(0)
        b = pl.program_id(1)
        s0 = j * blk
        ln = tl_ref[b]
        svec = s0 + lax.broadcasted_iota(jnp.int32, (blk, 1), 0)
        sep = c_ref[0:1, :]
        pad = c_ref[1:2, :]
        gamma = c_ref[2:3, :]
        beta = c_ref[3:4, :]
        sel = jnp.where(svec < ln - 6, we_ref[...],
                        jnp.where(svec == ln - 6, avg_ref[0],
                                  jnp.where(svec == ln - 5, sep, pad)))
        x = sel + pe_ref[...]
        mu = jnp.mean(x, axis=1, keepdims=True)
        xc = x - mu
        var = jnp.mean(xc * xc, axis=1, keepdims=True)
        y = xc * lax.rsqrt(var + EPS)
        out_ref[...] = y * gamma + beta

    return pl.pallas_call(
        body,
        grid=grid,
        in_specs=[
            pl.BlockSpec(memory_space=pltpu.SMEM),
            pl.BlockSpec((blk, H), lambda j, b: (b * sb + j, 0)),
            pl.BlockSpec((1, 1, H), lambda j, b: (b, 0, 0)),
            pl.BlockSpec((blk, H), lambda j, b: (j, 0)),
            pl.BlockSpec((8, H), lambda j, b: (0, 0)),
        ],
        out_specs=pl.BlockSpec((blk, H), lambda j, b: (b * sb + j, 0)),
        out_shape=jax.ShapeDtypeStruct((n, H), jnp.float32),
    )(text_len, we_flat, avg.reshape(-1, 1, H), pe_plus, consts)


def kernel(input_ids, text_len, word_emb, pos_emb, type_emb, ln_gamma, ln_beta):
    b, s = input_ids.shape
    ids_flat = input_ids.reshape(-1).astype(jnp.int32)
    tl = text_len.astype(jnp.int32)
    pe_plus = pos_emb + type_emb[0][None, :]
    consts = jnp.concatenate(
        [word_emb[102:103], word_emb[0:1], ln_gamma[None, :], ln_beta[None, :],
         jnp.zeros((4, H), jnp.float32)], axis=0)
    we_flat, avg = _sc_gather(word_emb, ids_flat, tl)
    out = _tc_combine(we_flat, avg, tl, pe_plus, consts)
    return out.reshape(b, s, H)


# speedup vs baseline: 4.1383x; 2.2876x over previous
import functools

import jax
import jax.numpy as jnp
from jax import lax
from jax.experimental import pallas as pl
from jax.experimental.pallas import tpu as pltpu
from jax.experimental.pallas import tpu_sc as plsc

H = 768
S = 512
EPS = 1e-12

_NC = 2
_NS = 16
_NW = _NC * _NS


def _sc_gather(word_emb, ids_flat, text_len):
    n = ids_flat.shape[0]
    nb = text_len.shape[0]
    rpw = n // _NW
    bpw = nb // _NW
    g = 64
    mesh = plsc.VectorSubcoreMesh(core_axis_name="c", subcore_axis_name="s",
                                  num_cores=_NC, num_subcores=_NS)

    @functools.partial(
        pl.kernel,
        out_type=(jax.ShapeDtypeStruct((n, H), jnp.float32),
                  jax.ShapeDtypeStruct((nb, H), jnp.float32)),
        mesh=mesh,
        scratch_types=[
            pltpu.VMEM((g,), jnp.int32),
            pltpu.VMEM((g, H), jnp.float32),
            pltpu.VMEM((16,), jnp.int32),
            pltpu.VMEM((16,), jnp.int32),
            pltpu.VMEM((16, H), jnp.float32),
            pltpu.VMEM((16,), jnp.int32),
            pltpu.VMEM((H,), jnp.float32),
            pltpu.SemaphoreType.DMA,
        ],
    )
    def gather_kernel(table_hbm, idx_hbm, tl_hbm, out_hbm, avg_hbm,
                      idx_v, rows_v, wids_v, widx_v, wrows_v, tl_v, avg_v,
                      sem):
        wid = lax.axis_index("s") * _NC + lax.axis_index("c")
        base = wid * rpw

        def body(i, carry):
            off = pl.multiple_of(base + i * g, g)
            pltpu.sync_copy(idx_hbm.at[pl.ds(off, g)], idx_v)
            pltpu.async_copy(table_hbm.at[idx_v], rows_v, sem).wait()
            pltpu.sync_copy(rows_v, out_hbm.at[pl.ds(off, g)])
            return carry

        lax.fori_loop(0, rpw // g, body, 0)

        iota = lax.broadcasted_iota(jnp.int32, (16,), 0)
        pltpu.sync_copy(tl_hbm.at[pl.ds(wid * bpw, bpw)], tl_v.at[pl.ds(0, bpw)])
        tl_vec = tl_v[...]
        for j in range(bpw):
            b = wid * bpw + j
            ln = tl_vec[j]
            c = jnp.maximum(ln - 6, 0)
            c8 = jnp.minimum((c // 8) * 8, S - 16)
            pltpu.sync_copy(idx_hbm.at[pl.ds(b * S + c8, 16)], wids_v)
            gidx = jnp.minimum((c - c8) + iota, 15)
            widx_v[...] = wids_v[...].at[gidx].get(mode="promise_in_bounds")
            pltpu.async_copy(table_hbm.at[widx_v], wrows_v, sem).wait()
            nlast = ln - 2 - c

            def kbody(k, carry):
                koff = pl.multiple_of(k * 16, 16)
                acc = jnp.zeros((16,), jnp.float32)
                for j2 in range(5):
                    w = jnp.where(nlast >= j2, 0.2, 0.0)
                    acc = acc + wrows_v[j2, pl.ds(koff, 16)] * w
                avg_v[pl.ds(koff, 16)] = acc
                return carry

            lax.fori_loop(0, H // 16, kbody, 0)
            pltpu.sync_copy(avg_v, avg_hbm.at[b])

    return gather_kernel(word_emb, ids_flat, text_len)


def _tc_combine(we_flat, avg, text_len, pe_plus, consts):
    n = we_flat.shape[0]
    blk = S
    nb = n // S
    grid = (nb,)

    def body(tl_ref, we_ref, avg_ref, pe_ref, c_ref, out_ref):
        b = pl.program_id(0)
        ln = tl_ref[b]
        svec = lax.broadcasted_iota(jnp.int32, (blk, 1), 0)
        sep = c_ref[0:1, :]
        pad = c_ref[1:2, :]
        gamma = c_ref[2:3, :]
        beta = c_ref[3:4, :]
        sel = jnp.where(svec < ln - 6, we_ref[...],
                        jnp.where(svec == ln - 6, avg_ref[0],
                                  jnp.where(svec == ln - 5, sep, pad)))
        x = sel + pe_ref[...]
        mu = jnp.mean(x, axis=1, keepdims=True)
        xc = x - mu
        var = jnp.mean(xc * xc, axis=1, keepdims=True)
        y = xc * lax.rsqrt(var + EPS)
        out_ref[...] = y * gamma + beta

    return pl.pallas_call(
        body,
        grid=grid,
        in_specs=[
            pl.BlockSpec(memory_space=pltpu.SMEM),
            pl.BlockSpec((blk, H), lambda b: (b, 0)),
            pl.BlockSpec((1, 1, H), lambda b: (b, 0, 0)),
            pl.BlockSpec((blk, H), lambda b: (0, 0)),
            pl.BlockSpec((8, H), lambda b: (0, 0)),
        ],
        out_specs=pl.BlockSpec((blk, H), lambda b: (b, 0)),
        out_shape=jax.ShapeDtypeStruct((n, H), jnp.float32),
    )(text_len, we_flat, avg.reshape(-1, 1, H), pe_plus, consts)


def kernel(input_ids, text_len, word_emb, pos_emb, type_emb, ln_gamma, ln_beta):
    b, s = input_ids.shape
    ids_flat = input_ids.reshape(-1).astype(jnp.int32)
    tl = text_len.astype(jnp.int32)
    pe_plus = pos_emb + type_emb[0][None, :]
    consts = jnp.concatenate(
        [word_emb[102:103], word_emb[0:1], ln_gamma[None, :], ln_beta[None, :],
         jnp.zeros((4, H), jnp.float32)], axis=0)
    we_flat, avg = _sc_gather(word_emb, ids_flat, tl)
    out = _tc_combine(we_flat, avg, tl, pe_plus, consts)
    return out.reshape(b, s, H)


# speedup vs baseline: 4.8724x; 1.1774x over previous
import functools

import jax
import jax.numpy as jnp
from jax import lax
from jax.experimental import pallas as pl
from jax.experimental.pallas import tpu as pltpu
from jax.experimental.pallas import tpu_sc as plsc

H = 768
S = 512
EPS = 1e-12

_NC = 2
_NS = 16
_NW = _NC * _NS


def _sc_gather(word_emb, ids_flat, text_len):
    n = ids_flat.shape[0]
    nb = text_len.shape[0]
    rpw = n // _NW
    bpw = nb // _NW
    g = 64
    mesh = plsc.VectorSubcoreMesh(core_axis_name="c", subcore_axis_name="s",
                                  num_cores=_NC, num_subcores=_NS)

    @functools.partial(
        pl.kernel,
        out_type=(jax.ShapeDtypeStruct((n, H), jnp.float32),
                  jax.ShapeDtypeStruct((nb, H), jnp.float32)),
        mesh=mesh,
        scratch_types=[
            pltpu.VMEM((g,), jnp.int32),
            pltpu.VMEM((g, H), jnp.float32),
            pltpu.VMEM((16,), jnp.int32),
            pltpu.VMEM((16,), jnp.int32),
            pltpu.VMEM((16, H), jnp.float32),
            pltpu.VMEM((16,), jnp.int32),
            pltpu.VMEM((H,), jnp.float32),
            pltpu.SemaphoreType.DMA,
        ],
    )
    def gather_kernel(table_hbm, idx_hbm, tl_hbm, out_hbm, avg_hbm,
                      idx_v, rows_v, wids_v, widx_v, wrows_v, tl_v, avg_v,
                      sem):
        wid = lax.axis_index("s") * _NC + lax.axis_index("c")

        iota = lax.broadcasted_iota(jnp.int32, (16,), 0)
        pltpu.sync_copy(tl_hbm.at[pl.ds(wid * bpw, bpw)], tl_v.at[pl.ds(0, bpw)])
        tl_vec = tl_v[...]

        for j in range(bpw):
            ln_j = tl_vec[j]
            nch = (jnp.clip(ln_j - 1, 0, S) + (g - 1)) // g
            base_b = (wid * bpw + j) * S

            def body(i, carry):
                off = pl.multiple_of(base_b + i * g, g)
                pltpu.sync_copy(idx_hbm.at[pl.ds(off, g)], idx_v)
                pltpu.async_copy(table_hbm.at[idx_v], rows_v, sem).wait()
                pltpu.sync_copy(rows_v, out_hbm.at[pl.ds(off, g)])
                return carry

            lax.fori_loop(0, nch, body, 0)

        for j in range(bpw):
            b = wid * bpw + j
            ln = tl_vec[j]
            c = jnp.maximum(ln - 6, 0)
            c8 = jnp.minimum((c // 8) * 8, S - 16)
            pltpu.sync_copy(idx_hbm.at[pl.ds(b * S + c8, 16)], wids_v)
            gidx = jnp.minimum((c - c8) + iota, 15)
            widx_v[...] = wids_v[...].at[gidx].get(mode="promise_in_bounds")
            pltpu.async_copy(table_hbm.at[widx_v], wrows_v, sem).wait()
            nlast = ln - 2 - c

            def kbody(k, carry):
                koff = pl.multiple_of(k * 16, 16)
                acc = jnp.zeros((16,), jnp.float32)
                for j2 in range(5):
                    w = jnp.where(nlast >= j2, 0.2, 0.0)
                    acc = acc + wrows_v[j2, pl.ds(koff, 16)] * w
                avg_v[pl.ds(koff, 16)] = acc
                return carry

            lax.fori_loop(0, H // 16, kbody, 0)
            pltpu.sync_copy(avg_v, avg_hbm.at[b])

    return gather_kernel(word_emb, ids_flat, text_len)


def _tc_combine(we_flat, avg, text_len, pe_plus, consts):
    n = we_flat.shape[0]
    blk = S
    nb = n // S
    grid = (nb,)

    def body(tl_ref, we_ref, avg_ref, pe_ref, c_ref, out_ref):
        b = pl.program_id(0)
        ln = tl_ref[b]
        svec = lax.broadcasted_iota(jnp.int32, (blk, 1), 0)
        sep = c_ref[0:1, :]
        pad = c_ref[1:2, :]
        gamma = c_ref[2:3, :]
        beta = c_ref[3:4, :]
        sel = jnp.where(svec < ln - 6, we_ref[...],
                        jnp.where(svec == ln - 6, avg_ref[0],
                                  jnp.where(svec == ln - 5, sep, pad)))
        x = sel + pe_ref[...]
        mu = jnp.mean(x, axis=1, keepdims=True)
        xc = x - mu
        var = jnp.mean(xc * xc, axis=1, keepdims=True)
        y = xc * lax.rsqrt(var + EPS)
        out_ref[...] = y * gamma + beta

    return pl.pallas_call(
        body,
        grid=grid,
        in_specs=[
            pl.BlockSpec(memory_space=pltpu.SMEM),
            pl.BlockSpec((blk, H), lambda b: (b, 0)),
            pl.BlockSpec((1, 1, H), lambda b: (b, 0, 0)),
            pl.BlockSpec((blk, H), lambda b: (0, 0)),
            pl.BlockSpec((8, H), lambda b: (0, 0)),
        ],
        out_specs=pl.BlockSpec((blk, H), lambda b: (b, 0)),
        out_shape=jax.ShapeDtypeStruct((n, H), jnp.float32),
    )(text_len, we_flat, avg.reshape(-1, 1, H), pe_plus, consts)


def kernel(input_ids, text_len, word_emb, pos_emb, type_emb, ln_gamma, ln_beta):
    b, s = input_ids.shape
    ids_flat = input_ids.reshape(-1).astype(jnp.int32)
    tl = text_len.astype(jnp.int32)
    pe_plus = pos_emb + type_emb[0][None, :]
    consts = jnp.concatenate(
        [word_emb[102:103], word_emb[0:1], ln_gamma[None, :], ln_beta[None, :],
         jnp.zeros((4, H), jnp.float32)], axis=0)
    we_flat, avg = _sc_gather(word_emb, ids_flat, tl)
    out = _tc_combine(we_flat, avg, tl, pe_plus, consts)
    return out.reshape(b, s, H)


# speedup vs baseline: 5.0938x; 1.0454x over previous
import functools

import jax
import jax.numpy as jnp
from jax import lax
from jax.experimental import pallas as pl
from jax.experimental.pallas import tpu as pltpu
from jax.experimental.pallas import tpu_sc as plsc

H = 768
S = 512
EPS = 1e-12

_NC = 2
_NS = 16
_NW = _NC * _NS


def _sc_gather(word_emb, ids_flat, text_len):
    n = ids_flat.shape[0]
    nb = text_len.shape[0]
    rpw = n // _NW
    bpw = nb // _NW
    g = 64
    mesh = plsc.VectorSubcoreMesh(core_axis_name="c", subcore_axis_name="s",
                                  num_cores=_NC, num_subcores=_NS)

    @functools.partial(
        pl.kernel,
        out_type=(jax.ShapeDtypeStruct((n, H), jnp.float32),
                  jax.ShapeDtypeStruct((nb, H), jnp.float32)),
        mesh=mesh,
        scratch_types=[
            pltpu.VMEM((g,), jnp.int32),
            pltpu.VMEM((g, H), jnp.float32),
            pltpu.VMEM((16,), jnp.int32),
            pltpu.VMEM((16,), jnp.int32),
            pltpu.VMEM((16, H), jnp.float32),
            pltpu.VMEM((16,), jnp.int32),
            pltpu.VMEM((H,), jnp.float32),
            pltpu.SemaphoreType.DMA,
        ],
    )
    def gather_kernel(table_hbm, idx_hbm, tl_hbm, out_hbm, avg_hbm,
                      idx_v, rows_v, wids_v, widx_v, wrows_v, tl_v, avg_v,
                      sem):
        wid = lax.axis_index("s") * _NC + lax.axis_index("c")

        iota = lax.broadcasted_iota(jnp.int32, (16,), 0)
        pltpu.sync_copy(tl_hbm.at[pl.ds(wid * bpw, bpw)], tl_v.at[pl.ds(0, bpw)])
        tl_vec = tl_v[...]

        for j in range(bpw):
            ln_j = tl_vec[j]
            nch = (jnp.clip(ln_j - 1, 0, S) + (g - 1)) // g
            base_b = (wid * bpw + j) * S

            def body(i, carry):
                off = pl.multiple_of(base_b + i * g, g)
                pltpu.sync_copy(idx_hbm.at[pl.ds(off, g)], idx_v)
                pltpu.async_copy(table_hbm.at[idx_v], rows_v, sem).wait()
                pltpu.sync_copy(rows_v, out_hbm.at[pl.ds(off, g)])
                return carry

            lax.fori_loop(0, nch, body, 0)

        for j in range(bpw):
            b = wid * bpw + j
            ln = tl_vec[j]
            c = jnp.maximum(ln - 6, 0)
            c8 = jnp.minimum((c // 8) * 8, S - 16)
            pltpu.sync_copy(idx_hbm.at[pl.ds(b * S + c8, 16)], wids_v)
            gidx = jnp.minimum((c - c8) + iota, 15)
            widx_v[...] = wids_v[...].at[gidx].get(mode="promise_in_bounds")
            pltpu.async_copy(table_hbm.at[widx_v], wrows_v, sem).wait()
            nlast = ln - 2 - c

            def kbody(k, carry):
                koff = pl.multiple_of(k * 16, 16)
                acc = jnp.zeros((16,), jnp.float32)
                for j2 in range(5):
                    w = jnp.where(nlast >= j2, 0.2, 0.0)
                    acc = acc + wrows_v[j2, pl.ds(koff, 16)] * w
                avg_v[pl.ds(koff, 16)] = acc
                return carry

            lax.fori_loop(0, H // 16, kbody, 0)
            pltpu.sync_copy(avg_v, avg_hbm.at[b])

    return gather_kernel(word_emb, ids_flat, text_len)


def _tc_combine(we_flat, avg, text_len, pe_plus, consts):
    n = we_flat.shape[0]
    blk = S
    nb = n // S
    grid = (nb,)

    nck = 8
    ck = S // nck

    def body(tl_ref, we_ref, avg_ref, pe_ref, c_ref, out_ref, web_ref, sem):
        b = pl.program_id(0)
        ln = tl_ref[b]

        def chunk_copies(bb, buf, do_start):
            lnb = tl_ref[bb]
            for k in range(nck):
                @pl.when(k * ck <= lnb - 2)
                def _():
                    cp = pltpu.make_async_copy(
                        we_ref.at[pl.ds(bb * S + k * ck, ck)],
                        buf.at[pl.ds(k * ck, ck)], sem)
                    if do_start:
                        cp.start()
                    else:
                        cp.wait()

        @pl.when(b == 0)
        def _():
            chunk_copies(0, web_ref.at[0], True)

        nxt = jnp.minimum(b + 1, nb - 1)

        @pl.when(b + 1 < nb)
        def _():
            chunk_copies(nxt, web_ref.at[(b + 1) % 2], True)

        chunk_copies(b, web_ref.at[b % 2], False)

        svec = lax.broadcasted_iota(jnp.int32, (blk, 1), 0)
        sep = c_ref[0:1, :]
        pad = c_ref[1:2, :]
        gamma = c_ref[2:3, :]
        beta = c_ref[3:4, :]
        sel = jnp.where(svec < ln - 6, web_ref[b % 2],
                        jnp.where(svec == ln - 6, avg_ref[0],
                                  jnp.where(svec == ln - 5, sep, pad)))
        x = sel + pe_ref[...]
        mu = jnp.mean(x, axis=1, keepdims=True)
        xc = x - mu
        var = jnp.mean(xc * xc, axis=1, keepdims=True)
        y = xc * lax.rsqrt(var + EPS)
        out_ref[...] = y * gamma + beta

    return pl.pallas_call(
        body,
        grid=grid,
        in_specs=[
            pl.BlockSpec(memory_space=pltpu.SMEM),
            pl.BlockSpec(memory_space=pl.ANY),
            pl.BlockSpec((1, 1, H), lambda b: (b, 0, 0)),
            pl.BlockSpec((blk, H), lambda b: (0, 0)),
            pl.BlockSpec((8, H), lambda b: (0, 0)),
        ],
        out_specs=pl.BlockSpec((blk, H), lambda b: (b, 0)),
        out_shape=jax.ShapeDtypeStruct((n, H), jnp.float32),
        scratch_shapes=[
            pltpu.VMEM((2, S, H), jnp.float32),
            pltpu.SemaphoreType.DMA,
        ],
    )(text_len, we_flat, avg.reshape(-1, 1, H), pe_plus, consts)


def kernel(input_ids, text_len, word_emb, pos_emb, type_emb, ln_gamma, ln_beta):
    b, s = input_ids.shape
    ids_flat = input_ids.reshape(-1).astype(jnp.int32)
    tl = text_len.astype(jnp.int32)
    pe_plus = pos_emb + type_emb[0][None, :]
    consts = jnp.concatenate(
        [word_emb[102:103], word_emb[0:1], ln_gamma[None, :], ln_beta[None, :],
         jnp.zeros((4, H), jnp.float32)], axis=0)
    we_flat, avg = _sc_gather(word_emb, ids_flat, tl)
    out = _tc_combine(we_flat, avg, tl, pe_plus, consts)
    return out.reshape(b, s, H)


# speedup vs baseline: 5.6044x; 1.1002x over previous
import functools

import jax
import jax.numpy as jnp
from jax import lax
from jax.experimental import pallas as pl
from jax.experimental.pallas import tpu as pltpu
from jax.experimental.pallas import tpu_sc as plsc

H = 768
S = 512
EPS = 1e-12

_NC = 2
_NS = 16
_NW = _NC * _NS


def _sc_gather(word_emb, ids_flat, text_len):
    n = ids_flat.shape[0]
    nb = text_len.shape[0]
    rpw = n // _NW
    bpw = nb // _NW
    g = 64
    mesh = plsc.VectorSubcoreMesh(core_axis_name="c", subcore_axis_name="s",
                                  num_cores=_NC, num_subcores=_NS)

    @functools.partial(
        pl.kernel,
        out_type=(jax.ShapeDtypeStruct((n, H), jnp.float32),
                  jax.ShapeDtypeStruct((nb, H), jnp.float32)),
        mesh=mesh,
        scratch_types=[
            pltpu.VMEM((S * (nb // _NW),), jnp.int32),
            pltpu.VMEM((2, g, H), jnp.float32),
            pltpu.VMEM((16,), jnp.int32),
            pltpu.VMEM((16,), jnp.int32),
            pltpu.VMEM((16, H), jnp.float32),
            pltpu.VMEM((16,), jnp.int32),
            pltpu.VMEM((H,), jnp.float32),
            pltpu.SemaphoreType.DMA,
            pltpu.SemaphoreType.DMA,
        ],
    )
    def gather_kernel(table_hbm, idx_hbm, tl_hbm, out_hbm, avg_hbm,
                      idxall_v, rows2_v, wids_v, widx_v, wrows_v, tl_v, avg_v,
                      sem, sem_w):
        wid = lax.axis_index("s") * _NC + lax.axis_index("c")

        iota = lax.broadcasted_iota(jnp.int32, (16,), 0)
        pltpu.sync_copy(tl_hbm.at[pl.ds(wid * bpw, bpw)], tl_v.at[pl.ds(0, bpw)])
        tl_vec = tl_v[...]
        base_w = wid * rpw
        pltpu.sync_copy(idx_hbm.at[pl.ds(base_w, rpw)], idxall_v)

        for j in range(bpw):
            ln_j = tl_vec[j]
            nch = (jnp.clip(ln_j - 1, 0, S) + (g - 1)) // g
            base_b = (wid * bpw + j) * S
            loc_b = j * S

            def body(i, carry):
                cur = i % 2
                off = pl.multiple_of(base_b + i * g, g)
                loff = pl.multiple_of(loc_b + i * g, g)

                @pl.when(i >= 2)
                def _():
                    pltpu.make_async_copy(
                        rows2_v.at[cur], out_hbm.at[pl.ds(off, g)],
                        sem_w).wait()

                pltpu.async_copy(
                    table_hbm.at[idxall_v.at[pl.ds(loff, g)]],
                    rows2_v.at[cur], sem).wait()
                pltpu.make_async_copy(
                    rows2_v.at[cur], out_hbm.at[pl.ds(off, g)], sem_w).start()
                return carry

            lax.fori_loop(0, nch, body, 0)

            for d in range(2):
                @pl.when(nch >= d + 1)
                def _():
                    pltpu.make_async_copy(
                        rows2_v.at[0], out_hbm.at[pl.ds(base_b, g)],
                        sem_w).wait()

        for j in range(bpw):
            b = wid * bpw + j
            ln = tl_vec[j]
            c = jnp.maximum(ln - 6, 0)
            c8 = jnp.minimum((c // 8) * 8, S - 16)
            pltpu.sync_copy(idx_hbm.at[pl.ds(b * S + c8, 16)], wids_v)
            gidx = jnp.minimum((c - c8) + iota, 15)
            widx_v[...] = wids_v[...].at[gidx].get(mode="promise_in_bounds")
            pltpu.async_copy(table_hbm.at[widx_v], wrows_v, sem).wait()
            nlast = ln - 2 - c

            def kbody(k, carry):
                koff = pl.multiple_of(k * 16, 16)
                acc = jnp.zeros((16,), jnp.float32)
                for j2 in range(5):
                    w = jnp.where(nlast >= j2, 0.2, 0.0)
                    acc = acc + wrows_v[j2, pl.ds(koff, 16)] * w
                avg_v[pl.ds(koff, 16)] = acc
                return carry

            lax.fori_loop(0, H // 16, kbody, 0)
            pltpu.sync_copy(avg_v, avg_hbm.at[b])

    return gather_kernel(word_emb, ids_flat, text_len)


def _tc_combine(we_flat, avg, text_len, pe_plus, consts):
    n = we_flat.shape[0]
    blk = S
    nb = n // S
    grid = (nb,)

    nck = 8
    ck = S // nck

    def body(tl_ref, we_ref, avg_ref, pe_ref, c_ref, out_ref, web_ref, sem):
        b = pl.program_id(0)
        ln = tl_ref[b]

        def chunk_copies(bb, buf, do_start):
            lnb = tl_ref[bb]
            for k in range(nck):
                @pl.when(k * ck <= lnb - 2)
                def _():
                    cp = pltpu.make_async_copy(
                        we_ref.at[pl.ds(bb * S + k * ck, ck)],
                        buf.at[pl.ds(k * ck, ck)], sem)
                    if do_start:
                        cp.start()
                    else:
                        cp.wait()

        @pl.when(b == 0)
        def _():
            chunk_copies(0, web_ref.at[0], True)

        nxt = jnp.minimum(b + 1, nb - 1)

        @pl.when(b + 1 < nb)
        def _():
            chunk_copies(nxt, web_ref.at[(b + 1) % 2], True)

        chunk_copies(b, web_ref.at[b % 2], False)

        svec = lax.broadcasted_iota(jnp.int32, (blk, 1), 0)
        sep = c_ref[0:1, :]
        pad = c_ref[1:2, :]
        gamma = c_ref[2:3, :]
        beta = c_ref[3:4, :]
        sel = jnp.where(svec < ln - 6, web_ref[b % 2],
                        jnp.where(svec == ln - 6, avg_ref[0],
                                  jnp.where(svec == ln - 5, sep, pad)))
        x = sel + pe_ref[...]
        mu = jnp.mean(x, axis=1, keepdims=True)
        xc = x - mu
        var = jnp.mean(xc * xc, axis=1, keepdims=True)
        y = xc * lax.rsqrt(var + EPS)
        out_ref[...] = y * gamma + beta

    return pl.pallas_call(
        body,
        grid=grid,
        in_specs=[
            pl.BlockSpec(memory_space=pltpu.SMEM),
            pl.BlockSpec(memory_space=pl.ANY),
            pl.BlockSpec((1, 1, H), lambda b: (b, 0, 0)),
            pl.BlockSpec((blk, H), lambda b: (0, 0)),
            pl.BlockSpec((8, H), lambda b: (0, 0)),
        ],
        out_specs=pl.BlockSpec((blk, H), lambda b: (b, 0)),
        out_shape=jax.ShapeDtypeStruct((n, H), jnp.float32),
        scratch_shapes=[
            pltpu.VMEM((2, S, H), jnp.float32),
            pltpu.SemaphoreType.DMA,
        ],
    )(text_len, we_flat, avg.reshape(-1, 1, H), pe_plus, consts)


def kernel(input_ids, text_len, word_emb, pos_emb, type_emb, ln_gamma, ln_beta):
    b, s = input_ids.shape
    ids_flat = input_ids.reshape(-1).astype(jnp.int32)
    tl = text_len.astype(jnp.int32)
    pe_plus = pos_emb + type_emb[0][None, :]
    consts = jnp.concatenate(
        [word_emb[102:103], word_emb[0:1], ln_gamma[None, :], ln_beta[None, :],
         jnp.zeros((4, H), jnp.float32)], axis=0)
    we_flat, avg = _sc_gather(word_emb, ids_flat, tl)
    out = _tc_combine(we_flat, avg, tl, pe_plus, consts)
    return out.reshape(b, s, H)
